# baseline (speedup 1.0000x reference)
import functools

import jax
import jax.numpy as jnp
from jax import lax
from jax.experimental import pallas as pl
from jax.experimental.pallas import tpu as pltpu
from jax.experimental.pallas import tpu_sc as plsc


T_SC = 3072
BLOCK_TC = 1024
NW = 32
TOKW = T_SC // NW
SLAB = 48
NG = 3
DHALF = 1024


def _tc_gate(x_ref, wt_ref, b_ref, out_ref):
    logits = jnp.dot(x_ref[...], wt_ref[...],
                     preferred_element_type=jnp.float32) + b_ref[...]
    m = jnp.max(logits, axis=-1, keepdims=True)
    e = jnp.exp(logits - m)
    out_ref[...] = e / jnp.sum(e, axis=-1, keepdims=True)


def _sc_gate(t_tc, x_hbm, wt_hbm, b_hbm, out_hbm, wt_v, xbuf, obuf, b_v):
    wid = lax.axis_index("s") * 2 + lax.axis_index("c")
    base = t_tc + wid * TOKW
    pltpu.sync_copy(b_hbm, b_v)
    bvec = b_v[...]
    toks = [lax.iota(jnp.int32, 16) + g * 16 for g in range(NG)]

    for p in range(TOKW // SLAB):
        pltpu.sync_copy(x_hbm.at[pl.ds(base + p * SLAB, SLAB), :], xbuf)
        accs = [jnp.zeros((16,), jnp.float32) + bvec[c]
                for c in range(16) for _ in range(NG)]

        for h in range(2048 // DHALF):
            pltpu.sync_copy(wt_hbm.at[pl.ds(h * (DHALF // 8), DHALF // 8), :],
                            wt_v)

            def dim_body(i, accs, h=h):
                accs = list(accs)
                for sub in range(2):
                    dl = i * 2 + sub
                    d = h * DHALF + dl
                    dcol = jnp.full((16,), d, jnp.int32)
                    xg = [plsc.load_gather(xbuf, [toks[g], dcol])
                          for g in range(NG)]
                    wv = wt_v[dl // 8, pl.ds((dl % 8) * 16, 16)]
                    for c in range(16):
                        wc = wv[c]
                        for g in range(NG):
                            accs[c * NG + g] = accs[c * NG + g] + xg[g] * wc
                return tuple(accs)

            accs = list(lax.fori_loop(0, DHALF // 2, dim_body, tuple(accs)))

        for g in range(NG):
            ac = [accs[c * NG + g] for c in range(16)]
            m = ac[0]
            for c in range(1, 16):
                m = jnp.maximum(m, ac[c])
            e = [jnp.exp(a - m) for a in ac]
            s = e[0]
            for c in range(1, 16):
                s = s + e[c]
            inv = 1.0 / s
            row = p * SLAB + g * 16
            ridx = lax.iota(jnp.int32, 16) + row
            for c in range(16):
                plsc.store_scatter(obuf, [ridx, jnp.full((16,), c, jnp.int32)],
                                   e[c] * inv)

    pltpu.sync_copy(obuf, out_hbm.at[pl.ds(wid * TOKW, TOKW), :])


def kernel(payload_tensor, W, b):
    B, S, D = payload_tensor.shape
    C = W.shape[0]
    T = B * S
    t_tc = T - T_SC
    x2 = payload_tensor.reshape(T, D)
    wt = W.T
    wt_packed = W.T.reshape(D // 8, 8 * C)
    b2 = b.reshape(1, C)

    sc_call = functools.partial(
        pl.kernel,
        mesh=plsc.VectorSubcoreMesh(core_axis_name="c", subcore_axis_name="s"),
        out_type=jax.ShapeDtypeStruct((T_SC, C), jnp.float32),
        scratch_types=[
            pltpu.VMEM((DHALF // 8, 8 * C), jnp.float32),
            pltpu.VMEM((SLAB, D), jnp.float32),
            pltpu.VMEM((TOKW, C), jnp.float32),
            pltpu.VMEM((C,), jnp.float32),
        ],
        compiler_params=pltpu.CompilerParams(needs_layout_passes=False),
    )(functools.partial(_sc_gate, t_tc))
    out_sc = sc_call(x2, wt_packed, b)

    out_tc = pl.pallas_call(
        _tc_gate,
        grid=(t_tc // BLOCK_TC,),
        in_specs=[
            pl.BlockSpec((BLOCK_TC, D), lambda i: (i, 0)),
            pl.BlockSpec((D, C), lambda i: (0, 0)),
            pl.BlockSpec((1, C), lambda i: (0, 0)),
        ],
        out_specs=pl.BlockSpec((BLOCK_TC, C), lambda i: (i, 0)),
        out_shape=jax.ShapeDtypeStruct((t_tc, C), jnp.float32),
    )(x2, wt, b2)

    out = jnp.concatenate([out_tc, out_sc], axis=0)
    return out.reshape(B, S, C)


# speedup vs baseline: 1.7535x; 1.7535x over previous
import functools

import jax
import jax.numpy as jnp
from jax import lax
from jax.experimental import pallas as pl
from jax.experimental.pallas import tpu as pltpu
from jax.experimental.pallas import tpu_sc as plsc


T_SC = 2048
BLOCK_TC = 1024
NW = 32
TOKW = T_SC // NW
SLAB = 32
NG = 2
DHALF = 1024
XPAD = 8


def _tc_gate(x_ref, wt_ref, b_ref, out_ref):
    logits = jnp.dot(x_ref[...], wt_ref[...],
                     preferred_element_type=jnp.float32) + b_ref[...]
    m = jnp.max(logits, axis=-1, keepdims=True)
    e = jnp.exp(logits - m)
    out_ref[...] = e / jnp.sum(e, axis=-1, keepdims=True)


def _sc_gate(t_tc, x_hbm, wt_hbm, b_hbm, out_hbm, wt_v, xbuf, obuf, b_v):
    wid = lax.axis_index("s") * 2 + lax.axis_index("c")
    base = t_tc + wid * TOKW
    pltpu.sync_copy(b_hbm, b_v)
    bvec = b_v[...]
    toks = [lax.iota(jnp.int32, 16) + g * 16 for g in range(NG)]

    for p in range(TOKW // SLAB):
        pltpu.sync_copy(x_hbm.at[pl.ds(base + p * SLAB, SLAB), :],
                        xbuf.at[:, pl.ds(0, 2048)])
        accs = [jnp.zeros((16,), jnp.float32) + bvec[c]
                for c in range(16) for _ in range(NG)]

        for h in range(2048 // DHALF):
            pltpu.sync_copy(wt_hbm.at[pl.ds(h * (DHALF // 8), DHALF // 8), :],
                            wt_v)

            def dim_body(i, accs, h=h):
                accs = list(accs)
                for sub in range(2):
                    dl = i * 2 + sub
                    d = h * DHALF + dl
                    dcol = jnp.full((16,), d, jnp.int32)
                    xg = [plsc.load_gather(xbuf, [toks[g], dcol])
                          for g in range(NG)]
                    wv = wt_v[dl // 8, pl.ds((dl % 8) * 16, 16)]
                    for c in range(16):
                        wc = wv[c]
                        for g in range(NG):
                            accs[c * NG + g] = accs[c * NG + g] + xg[g] * wc
                return tuple(accs)

            accs = list(lax.fori_loop(0, DHALF // 2, dim_body, tuple(accs)))

        for g in range(NG):
            ac = [accs[c * NG + g] for c in range(16)]
            m = ac[0]
            for c in range(1, 16):
                m = jnp.maximum(m, ac[c])
            e = [jnp.exp(a - m) for a in ac]
            s = e[0]
            for c in range(1, 16):
                s = s + e[c]
            inv = 1.0 / s
            row = p * SLAB + g * 16
            ridx = lax.iota(jnp.int32, 16) + row
            for c in range(16):
                plsc.store_scatter(obuf, [ridx, jnp.full((16,), c, jnp.int32)],
                                   e[c] * inv)

    pltpu.sync_copy(obuf, out_hbm.at[pl.ds(wid * TOKW, TOKW), :])


def kernel(payload_tensor, W, b):
    B, S, D = payload_tensor.shape
    C = W.shape[0]
    T = B * S
    t_tc = T - T_SC
    x2 = payload_tensor.reshape(T, D)
    wt = W.T
    wt_packed = W.T.reshape(D // 8, 8 * C)
    b2 = b.reshape(1, C)

    sc_call = functools.partial(
        pl.kernel,
        mesh=plsc.VectorSubcoreMesh(core_axis_name="c", subcore_axis_name="s"),
        out_type=jax.ShapeDtypeStruct((T_SC, C), jnp.float32),
        scratch_types=[
            pltpu.VMEM((DHALF // 8, 8 * C), jnp.float32),
            pltpu.VMEM((SLAB, D + XPAD), jnp.float32),
            pltpu.VMEM((TOKW, C), jnp.float32),
            pltpu.VMEM((C,), jnp.float32),
        ],
        compiler_params=pltpu.CompilerParams(needs_layout_passes=False),
    )(functools.partial(_sc_gate, t_tc))
    out_sc = sc_call(x2, wt_packed, b)

    out_tc = pl.pallas_call(
        _tc_gate,
        grid=(t_tc // BLOCK_TC,),
        in_specs=[
            pl.BlockSpec((BLOCK_TC, D), lambda i: (i, 0)),
            pl.BlockSpec((D, C), lambda i: (0, 0)),
            pl.BlockSpec((1, C), lambda i: (0, 0)),
        ],
        out_specs=pl.BlockSpec((BLOCK_TC, C), lambda i: (i, 0)),
        out_shape=jax.ShapeDtypeStruct((t_tc, C), jnp.float32),
    )(x2, wt, b2)

    out = jnp.concatenate([out_tc, out_sc], axis=0)
    return out.reshape(B, S, C)


# speedup vs baseline: 7.3236x; 4.1766x over previous
import jax
import jax.numpy as jnp
from jax.experimental import pallas as pl


BLOCK_TOKENS = 2048


def _gate_kernel(x_ref, wt_ref, b_ref, out_ref):
    x = x_ref[...]
    wt = wt_ref[...]
    logits = jnp.dot(x, wt, preferred_element_type=jnp.float32) + b_ref[...]
    m = jnp.max(logits, axis=-1, keepdims=True)
    e = jnp.exp(logits - m)
    out_ref[...] = e / jnp.sum(e, axis=-1, keepdims=True)


def kernel(payload_tensor, W, b):
    B, S, D = payload_tensor.shape
    C = W.shape[0]
    T = B * S
    x2 = payload_tensor.reshape(T, D)
    wt = W.T
    b2 = b.reshape(1, C)

    grid = (T // BLOCK_TOKENS,)
    out = pl.pallas_call(
        _gate_kernel,
        grid=grid,
        in_specs=[
            pl.BlockSpec((BLOCK_TOKENS, D), lambda i: (i, 0)),
            pl.BlockSpec((D, C), lambda i: (0, 0)),
            pl.BlockSpec((1, C), lambda i: (0, 0)),
        ],
        out_specs=pl.BlockSpec((BLOCK_TOKENS, C), lambda i: (i, 0)),
        out_shape=jax.ShapeDtypeStruct((T, C), jnp.float32),
    )(x2, wt, b2)
    return out.reshape(B, S, C)
